# baseline (speedup 1.0000x reference)
import functools

import jax
import jax.numpy as jnp
from jax import lax
from jax.experimental import pallas as pl
from jax.experimental.pallas import tpu as pltpu, tpu_sc as plsc

_M, _N = 16384, 1024
_TOTAL = _M * _N
_NC, _NS, _L = 2, 16, 16
_NW = _NC * _NS
_PER_W = _TOTAL // _NW
_CHUNK = 65536
_NCHUNKS = _PER_W // _CHUNK
_VECS = _CHUNK // _L


def _sc_body(x_hbm, o_hbm, buf, sem):
    wid = lax.axis_index("s") * _NC + lax.axis_index("c")
    base = wid * _PER_W

    @pl.loop(0, _NCHUNKS)
    def _chunks(c):
        off = base + c * _CHUNK
        pltpu.async_copy(x_hbm.at[pl.ds(off, _CHUNK)], buf, sem).wait()

        @pl.loop(0, _VECS, unroll=8)
        def _vecs(i):
            v = buf[pl.ds(i * _L, _L)]
            buf[pl.ds(i * _L, _L)] = v + v - 1.0

        pltpu.async_copy(buf, o_hbm.at[pl.ds(off, _CHUNK)], sem).wait()


@jax.jit
def kernel(x):
    mesh = plsc.VectorSubcoreMesh(core_axis_name="c", subcore_axis_name="s")
    flat = x.reshape(_TOTAL)
    out = pl.kernel(
        _sc_body,
        out_type=jax.ShapeDtypeStruct((_TOTAL,), jnp.float32),
        mesh=mesh,
        scratch_types=[
            pltpu.VMEM((_CHUNK,), jnp.float32),
            pltpu.SemaphoreType.DMA,
        ],
    )(flat)
    return out.reshape(_M, _N)


# speedup vs baseline: 1.1281x; 1.1281x over previous
import jax
import jax.numpy as jnp
from jax import lax
from jax.experimental import pallas as pl
from jax.experimental.pallas import tpu as pltpu, tpu_sc as plsc

_M, _N = 16384, 1024
_TOTAL = _M * _N
_NC, _NS, _L = 2, 16, 16
_NW = _NC * _NS
_PER_W = _TOTAL // _NW
_CHUNK = 32768
_NCHUNKS = _PER_W // _CHUNK
_VECS = _CHUNK // _L


def _sc_body(x_hbm, o_hbm, b0, b1, is0, is1, os0, os1):
    bufs = (b0, b1)
    isems = (is0, is1)
    osems = (os0, os1)
    wid = lax.axis_index("s") * _NC + lax.axis_index("c")
    base = wid * _PER_W

    def in_slice(c):
        return x_hbm.at[pl.ds(base + c * _CHUNK, _CHUNK)]

    def out_slice(c):
        return o_hbm.at[pl.ds(base + c * _CHUNK, _CHUNK)]

    pltpu.async_copy(in_slice(0), bufs[0], isems[0])
    for c in range(_NCHUNKS):
        b = c % 2
        nb = 1 - b
        pltpu.make_async_copy(in_slice(c), bufs[b], isems[b]).wait()
        if c + 1 < _NCHUNKS:
            if c >= 1:
                pltpu.make_async_copy(bufs[nb], out_slice(c - 1), osems[nb]).wait()
            pltpu.async_copy(in_slice(c + 1), bufs[nb], isems[nb])

        @pl.loop(0, _VECS, unroll=8)
        def _vecs(i, buf=bufs[b]):
            v = buf[pl.ds(i * _L, _L)]
            buf[pl.ds(i * _L, _L)] = v + v - 1.0

        pltpu.async_copy(bufs[b], out_slice(c), osems[b])
    pltpu.make_async_copy(bufs[0], out_slice(_NCHUNKS - 2), osems[0]).wait()
    pltpu.make_async_copy(bufs[1], out_slice(_NCHUNKS - 1), osems[1]).wait()


@jax.jit
def kernel(x):
    mesh = plsc.VectorSubcoreMesh(core_axis_name="c", subcore_axis_name="s")
    flat = x.reshape(_TOTAL)
    out = pl.kernel(
        _sc_body,
        out_type=jax.ShapeDtypeStruct((_TOTAL,), jnp.float32),
        mesh=mesh,
        scratch_types=[
            pltpu.VMEM((_CHUNK,), jnp.float32),
            pltpu.VMEM((_CHUNK,), jnp.float32),
            pltpu.SemaphoreType.DMA,
            pltpu.SemaphoreType.DMA,
            pltpu.SemaphoreType.DMA,
            pltpu.SemaphoreType.DMA,
        ],
    )(flat)
    return out.reshape(_M, _N)


# speedup vs baseline: 1.1307x; 1.0022x over previous
import jax
import jax.numpy as jnp
from jax import lax
from jax.experimental import pallas as pl
from jax.experimental.pallas import tpu as pltpu, tpu_sc as plsc

_M, _N = 16384, 1024
_TOTAL = _M * _N
_NC, _NS, _L = 2, 16, 16
_NW = _NC * _NS
_PER_W = _TOTAL // _NW
_CHUNK = 32768
_NCHUNKS = _PER_W // _CHUNK
_VECS = _CHUNK // _L


def _sc_body(x_hbm, o_hbm, b0, b1, is0, is1, os0, os1):
    bufs = (b0, b1)
    isems = (is0, is1)
    osems = (os0, os1)
    wid = lax.axis_index("s") * _NC + lax.axis_index("c")
    base = wid * _PER_W

    def in_slice(c):
        return x_hbm.at[pl.ds(base + c * _CHUNK, _CHUNK)]

    def out_slice(c):
        return o_hbm.at[pl.ds(base + c * _CHUNK, _CHUNK)]

    pltpu.async_copy(in_slice(0), bufs[0], isems[0])
    for c in range(_NCHUNKS):
        b = c % 2
        nb = 1 - b
        pltpu.make_async_copy(in_slice(c), bufs[b], isems[b]).wait()
        if c + 1 < _NCHUNKS:
            if c >= 1:
                pltpu.make_async_copy(bufs[nb], out_slice(c - 1), osems[nb]).wait()
            pltpu.async_copy(in_slice(c + 1), bufs[nb], isems[nb])

        @plsc.parallel_loop(0, _VECS, unroll=8)
        def _vecs(i, buf=bufs[b]):
            v = buf[pl.ds(i * _L, _L)]
            buf[pl.ds(i * _L, _L)] = v + v - 1.0

        pltpu.async_copy(bufs[b], out_slice(c), osems[b])
    pltpu.make_async_copy(bufs[0], out_slice(_NCHUNKS - 2), osems[0]).wait()
    pltpu.make_async_copy(bufs[1], out_slice(_NCHUNKS - 1), osems[1]).wait()


@jax.jit
def kernel(x):
    mesh = plsc.VectorSubcoreMesh(core_axis_name="c", subcore_axis_name="s")
    flat = x.reshape(_TOTAL)
    out = pl.kernel(
        _sc_body,
        out_type=jax.ShapeDtypeStruct((_TOTAL,), jnp.float32),
        mesh=mesh,
        scratch_types=[
            pltpu.VMEM((_CHUNK,), jnp.float32),
            pltpu.VMEM((_CHUNK,), jnp.float32),
            pltpu.SemaphoreType.DMA,
            pltpu.SemaphoreType.DMA,
            pltpu.SemaphoreType.DMA,
            pltpu.SemaphoreType.DMA,
        ],
    )(flat)
    return out.reshape(_M, _N)


# speedup vs baseline: 3.0695x; 2.7148x over previous
import jax
import jax.numpy as jnp
from jax import lax
from jax.experimental import pallas as pl
from jax.experimental.pallas import tpu as pltpu, tpu_sc as plsc

_M, _N = 16384, 1024
_NC, _NS, _L = 2, 16, 16
_NW = _NC * _NS
_ROWS_W = _M // _NW
_CR = 32
_NCHUNKS = _ROWS_W // _CR
_VPR = _N // _L


def _sc_body(x_hbm, o_hbm, b0, b1, is0, is1, os0, os1):
    bufs = (b0, b1)
    isems = (is0, is1)
    osems = (os0, os1)
    wid = lax.axis_index("s") * _NC + lax.axis_index("c")
    base = wid * _ROWS_W

    def in_slice(c):
        return x_hbm.at[pl.ds(base + c * _CR, _CR), :]

    def out_slice(c):
        return o_hbm.at[pl.ds(base + c * _CR, _CR), :]

    pltpu.async_copy(in_slice(0), bufs[0], isems[0])
    for c in range(_NCHUNKS):
        b = c % 2
        nb = 1 - b
        pltpu.make_async_copy(in_slice(c), bufs[b], isems[b]).wait()
        if c + 1 < _NCHUNKS:
            if c >= 1:
                pltpu.make_async_copy(bufs[nb], out_slice(c - 1), osems[nb]).wait()
            pltpu.async_copy(in_slice(c + 1), bufs[nb], isems[nb])

        @pl.loop(0, _CR)
        def _rows(r, buf=bufs[b]):
            @plsc.parallel_loop(0, _VPR, unroll=8)
            def _vecs(j):
                v = buf[r, pl.ds(j * _L, _L)]
                buf[r, pl.ds(j * _L, _L)] = v + v - 1.0

        pltpu.async_copy(bufs[b], out_slice(c), osems[b])
    pltpu.make_async_copy(bufs[0], out_slice(_NCHUNKS - 2), osems[0]).wait()
    pltpu.make_async_copy(bufs[1], out_slice(_NCHUNKS - 1), osems[1]).wait()


@jax.jit
def kernel(x):
    mesh = plsc.VectorSubcoreMesh(core_axis_name="c", subcore_axis_name="s")
    return pl.kernel(
        _sc_body,
        out_type=jax.ShapeDtypeStruct((_M, _N), jnp.float32),
        mesh=mesh,
        scratch_types=[
            pltpu.VMEM((_CR, _N), jnp.float32),
            pltpu.VMEM((_CR, _N), jnp.float32),
            pltpu.SemaphoreType.DMA,
            pltpu.SemaphoreType.DMA,
            pltpu.SemaphoreType.DMA,
            pltpu.SemaphoreType.DMA,
        ],
    )(x)
